# Initial kernel scaffold; baseline (speedup 1.0000x reference)
import jax
import jax.numpy as jnp
from jax.experimental import pallas as pl


def kernel(syn_embeddings, herb_embeddings, norm_adj_idx, norm_adj_val, sub1_idx, sub1_val, sub2_idx, sub2_val, users, items, neg_items, alpha):
    raise NotImplementedError("write your pallas kernel here")


# speedup vs baseline: 2.1105x; 2.1105x over previous
import functools

import jax
import jax.numpy as jnp
from jax import lax
from jax.experimental import pallas as pl
from jax.experimental.pallas import tpu as pltpu
from jax.experimental.pallas import tpu_sc as plsc

SS_ = 4000
HH_ = 6000
N_ = SS_ + HH_
E_ = 320000
D_ = 128
B_ = 1024
TEMP = 0.2

NC = 2
NS = 16
NW = NC * NS
DH = D_ // NC
EPT = E_ // NS
CH = 80
NCH = EPT // CH
RPT = 624
VB = DH // 16


def _sc_mesh():
    return plsc.VectorSubcoreMesh(core_axis_name="c", subcore_axis_name="s")


def _propagate(ego_cols, src3, dst3, val3):
    @functools.partial(
        pl.kernel,
        out_type=jax.ShapeDtypeStruct((NC, N_, DH), jnp.float32),
        mesh=_sc_mesh(),
        compiler_params=pltpu.CompilerParams(use_tc_tiling_on_sc=False),
        scratch_types=[
            pltpu.VMEM((NCH, CH), jnp.int32),
            pltpu.VMEM((NCH, CH), jnp.int32),
            pltpu.VMEM((NCH, CH), jnp.float32),
            pltpu.VMEM((CH, DH), jnp.float32),
            pltpu.VMEM_SHARED((N_, DH), jnp.float32),
            pltpu.SemaphoreType.DMA,
        ],
    )
    def prop(ego_hbm, src_hbm, dst_hbm, val_hbm, out_hbm,
             src_v, dst_v, val_v, rows_v, acc_sh, sem):
        c = lax.axis_index("c")
        s = lax.axis_index("s")

        def _zrow(i, carry):
            for j in range(VB):
                rows_v[i, pl.ds(j * 16, 16)] = jnp.zeros((16,), jnp.float32)
            return carry
        lax.fori_loop(0, CH, _zrow, 0)

        base_r = s * RPT
        nfull = RPT // CH
        rem = RPT - nfull * CH

        def _zcopy(t, carry):
            pltpu.sync_copy(rows_v, acc_sh.at[pl.ds(base_r + t * CH, CH)])
            return carry
        lax.fori_loop(0, nfull, _zcopy, 0)
        pltpu.sync_copy(rows_v.at[pl.ds(0, rem)],
                        acc_sh.at[pl.ds(base_r + nfull * CH, rem)])

        @pl.when(s == NS - 1)
        def _ztail():
            pltpu.sync_copy(rows_v.at[pl.ds(0, N_ - NS * RPT)],
                            acc_sh.at[pl.ds(NS * RPT, N_ - NS * RPT)])
        plsc.subcore_barrier()

        pltpu.sync_copy(src_hbm.at[s], src_v)
        pltpu.sync_copy(dst_hbm.at[s], dst_v)
        pltpu.sync_copy(val_hbm.at[s], val_v)
        my_cols = ego_hbm.at[c]

        def _chunk(k, carry):
            pltpu.async_copy(my_cols.at[src_v.at[k]], rows_v, sem).wait()

            def _mul(i16, inner):
                vv = val_v[k, pl.ds(i16 * 16, 16)]
                for lane in range(16):
                    v = vv[lane]
                    row = i16 * 16 + lane
                    for j in range(VB):
                        sl = pl.ds(j * 16, 16)
                        rows_v[row, sl] = rows_v[row, sl] * v
                return inner
            lax.fori_loop(0, CH // 16, _mul, 0)

            pltpu.sync_copy(rows_v, acc_sh.at[dst_v.at[k]], add=True)
            return carry
        lax.fori_loop(0, NCH, _chunk, 0)
        plsc.subcore_barrier()

        pltpu.sync_copy(acc_sh.at[pl.ds(base_r, RPT)],
                        out_hbm.at[c, pl.ds(base_r, RPT)])

        @pl.when(s == NS - 1)
        def _otail():
            pltpu.sync_copy(acc_sh.at[pl.ds(NS * RPT, N_ - NS * RPT)],
                            out_hbm.at[c, pl.ds(NS * RPT, N_ - NS * RPT)])

    return prop(ego_cols, src3, dst3, val3)


def _update(p, ego_cols, acc, a_row, scale):
    blk = 1000

    def body(p_ref, e_ref, ac_ref, a_ref, eo_ref, ao_ref):
        a = a_ref[...]
        neigh = jnp.concatenate([p_ref[0], p_ref[1]], axis=1)
        ego = jnp.concatenate([e_ref[0], e_ref[1]], axis=1)
        new = jnp.maximum(a * neigh + (1.0 - a) * ego, 0.0)
        eo_ref[0] = new[:, :DH]
        eo_ref[1] = new[:, DH:]
        ao_ref[...] = (ac_ref[...] + new) * scale

    return pl.pallas_call(
        body,
        grid=(N_ // blk,),
        in_specs=[
            pl.BlockSpec((NC, blk, DH), lambda i: (0, i, 0)),
            pl.BlockSpec((NC, blk, DH), lambda i: (0, i, 0)),
            pl.BlockSpec((blk, D_), lambda i: (i, 0)),
            pl.BlockSpec((1, D_), lambda i: (0, 0)),
        ],
        out_specs=[
            pl.BlockSpec((NC, blk, DH), lambda i: (0, i, 0)),
            pl.BlockSpec((blk, D_), lambda i: (i, 0)),
        ],
        out_shape=[
            jax.ShapeDtypeStruct((NC, N_, DH), jnp.float32),
            jax.ShapeDtypeStruct((N_, D_), jnp.float32),
        ],
    )(p, ego_cols, acc, a_row)


def _gather7(allm, allm1, allm2, users, items_off, neg_off):
    RG = B_ // NW

    @functools.partial(
        pl.kernel,
        out_type=tuple(jax.ShapeDtypeStruct((B_, D_), jnp.float32)
                       for _ in range(7)),
        mesh=_sc_mesh(),
        scratch_types=[
            pltpu.VMEM((RG,), jnp.int32),
            pltpu.VMEM((RG,), jnp.int32),
            pltpu.VMEM((RG,), jnp.int32),
            pltpu.VMEM((RG, D_), jnp.float32),
        ],
    )
    def gk(t0, t1, t2, u_hbm, i_hbm, n_hbm,
           o_ue, o_ie, o_ine, o_u1, o_i1, o_u2, o_i2,
           iu, ii, inn, rows):
        c = lax.axis_index("c")
        s = lax.axis_index("s")
        b = (s * NC + c) * RG
        pltpu.sync_copy(u_hbm.at[pl.ds(b, RG)], iu)
        pltpu.sync_copy(i_hbm.at[pl.ds(b, RG)], ii)
        pltpu.sync_copy(n_hbm.at[pl.ds(b, RG)], inn)
        for tab, idx, out in ((t0, iu, o_ue), (t0, ii, o_ie), (t0, inn, o_ine),
                              (t1, iu, o_u1), (t1, ii, o_i1),
                              (t2, iu, o_u2), (t2, ii, o_i2)):
            pltpu.sync_copy(tab.at[idx], rows)
            pltpu.sync_copy(rows, out.at[pl.ds(b, RG)])

    return gk(allm, allm1, allm2, users, items_off, neg_off)


def _norm_rows(x):
    return x / (jnp.sqrt(jnp.sum(x * x, axis=1, keepdims=True)) + 1e-12)


def _sup(gu, gi, gn):
    def body(u_ref, i_ref, n_ref, o_ref):
        u = u_ref[...]
        o_ref[...] = jnp.sum(u * (i_ref[...] - n_ref[...]), axis=1,
                             keepdims=True)
    return pl.pallas_call(
        body,
        out_shape=jax.ShapeDtypeStruct((B_, 1), jnp.float32),
    )(gu, gi, gn)


def _ssl(g1, g2, tab):
    t_rows = tab.shape[0]

    def body(g1_ref, g2_ref, t_ref, o_ref):
        z1 = _norm_rows(g1_ref[...])
        z2 = _norm_rows(g2_ref[...])
        pos = jnp.sum(z1 * z2, axis=1, keepdims=True)
        tn = _norm_rows(t_ref[...])
        o_ref[...] = lax.dot_general(
            z1, tn, (((1,), (1,)), ((), ())),
            preferred_element_type=jnp.float32) - pos

    return pl.pallas_call(
        body,
        out_shape=jax.ShapeDtypeStruct((B_, t_rows), jnp.float32),
    )(g1, g2, tab)


def _nce(gu1, gu2, gi1, gi2):
    def one(a, b):
        z1 = _norm_rows(a)
        z2 = _norm_rows(b)
        neg = lax.dot_general(z1, z2, (((1,), (1,)), ((), ())),
                              preferred_element_type=jnp.float32) / TEMP
        pos = jnp.sum(z1 * z2, axis=1) / TEMP
        m = jnp.max(neg, axis=1)
        lse = jnp.log(jnp.sum(jnp.exp(neg - m[:, None]), axis=1)) + m
        return jnp.mean(lse - pos)

    def body(u1_ref, u2_ref, i1_ref, i2_ref, o_ref):
        o_ref[...] = jnp.reshape(
            one(u1_ref[...], u2_ref[...]) + one(i1_ref[...], i2_ref[...]),
            (1, 1))

    return pl.pallas_call(
        body,
        out_shape=jax.ShapeDtypeStruct((1, 1), jnp.float32),
    )(gu1, gu2, gi1, gi2)


def kernel(syn_embeddings, herb_embeddings, norm_adj_idx, norm_adj_val,
           sub1_idx, sub1_val, sub2_idx, sub2_val,
           users, items, neg_items, alpha):
    ego0 = jnp.concatenate([syn_embeddings, herb_embeddings], axis=0)
    ego0_cols = jnp.stack([ego0[:, :DH], ego0[:, DH:]], axis=0)
    a = jax.nn.sigmoid(alpha).astype(jnp.float32)
    a_row = jnp.broadcast_to(a, (1, D_))

    def run_graph(idx, val):
        src3 = idx[1].reshape(NS, NCH, CH)
        dst3 = idx[0].reshape(NS, NCH, CH)
        val3 = val.reshape(NS, NCH, CH)
        ego = ego0_cols
        acc = ego0
        for layer in range(3):
            p = _propagate(ego, src3, dst3, val3)
            scale = 0.25 if layer == 2 else 1.0
            ego, acc = _update(p, ego, acc, a_row, scale)
        return acc

    allm = run_graph(norm_adj_idx, norm_adj_val)
    allm1 = run_graph(sub1_idx, sub1_val)
    allm2 = run_graph(sub2_idx, sub2_val)

    items_off = items + SS_
    neg_off = neg_items + SS_
    g_ue, g_ie, g_ine, g_u1, g_i1, g_u2, g_i2 = _gather7(
        allm, allm1, allm2, users, items_off, neg_off)

    sup_logits = _sup(g_ue, g_ie, g_ine)[:, 0]
    ssl_logits_user = _ssl(g_u1, g_u2, allm2[:SS_])
    ssl_logits_item = _ssl(g_i1, g_i2, allm2[SS_:])
    infonce_loss = _nce(g_u1, g_u2, g_i1, g_i2)[0, 0]

    ue = allm[:SS_]
    ie = allm[SS_:]
    return (sup_logits, ssl_logits_user, ssl_logits_item, ue, ie,
            infonce_loss)


# speedup vs baseline: 7.2920x; 3.4550x over previous
import functools

import jax
import jax.numpy as jnp
from jax import lax
from jax.experimental import pallas as pl
from jax.experimental.pallas import tpu as pltpu
from jax.experimental.pallas import tpu_sc as plsc

SS_ = 4000
HH_ = 6000
N_ = SS_ + HH_
E_ = 320000
D_ = 128
B_ = 1024
TEMP = 0.2

NC = 2
NS = 16
NW = NC * NS
DH = D_ // NC
EPT = E_ // NS
CH = 80
NCH = EPT // CH
RPT = 624
VB = DH // 16


def _sc_mesh():
    return plsc.VectorSubcoreMesh(core_axis_name="c", subcore_axis_name="s")


NB = 2
NR = NCH // NB


def _propagate(ego_cols, src3, dst3, val3):
    @functools.partial(
        pl.kernel,
        out_type=jax.ShapeDtypeStruct((NC, N_, DH), jnp.float32),
        mesh=_sc_mesh(),
        compiler_params=pltpu.CompilerParams(use_tc_tiling_on_sc=False),
        scratch_types=[
            pltpu.VMEM((NCH, CH), jnp.int32),
            pltpu.VMEM((NCH, CH), jnp.int32),
            pltpu.VMEM((NCH, CH), jnp.float32),
            pltpu.VMEM((NB, CH, DH), jnp.float32),
            pltpu.VMEM((NB, CH, DH), jnp.float32),
            pltpu.VMEM_SHARED((N_, DH), jnp.float32),
        ] + [pltpu.SemaphoreType.DMA] * (2 * NB),
    )
    def prop(ego_hbm, src_hbm, dst_hbm, val_hbm, out_hbm,
             src_v, dst_v, val_v, rin, rout, acc_sh, *sems):
        gsem = sems[:NB]
        ssem = sems[NB:]
        c = lax.axis_index("c")
        s = lax.axis_index("s")
        my_cols = ego_hbm.at[c]

        def g_start(b, k):
            pltpu.async_copy(my_cols.at[src_v.at[k]], rin.at[b], gsem[b])

        def g_wait(b, k):
            pltpu.make_async_copy(my_cols.at[src_v.at[k]], rin.at[b],
                                  gsem[b]).wait()

        def s_start(b, k):
            pltpu.async_copy(rout.at[b], acc_sh.at[dst_v.at[k]], ssem[b],
                             add=True)

        def s_wait(b, k):
            pltpu.make_async_copy(rout.at[b], acc_sh.at[dst_v.at[k]],
                                  ssem[b]).wait()

        def mult(b, k):
            def _m(i16, inner):
                vv = val_v[k, pl.ds(i16 * 16, 16)]
                for lane in range(16):
                    v = vv[lane]
                    row = i16 * 16 + lane
                    for j in range(VB):
                        sl = pl.ds(j * 16, 16)
                        rout[b, row, sl] = rin[b, row, sl] * v
                return inner
            lax.fori_loop(0, CH // 16, _m, 0)

        def _zrow(i, carry):
            for j in range(VB):
                rin[0, i, pl.ds(j * 16, 16)] = jnp.zeros((16,), jnp.float32)
            return carry
        lax.fori_loop(0, CH, _zrow, 0)

        base_r = s * RPT
        nfull = RPT // CH
        rem = RPT - nfull * CH

        def _zcopy(t, carry):
            pltpu.sync_copy(rin.at[0], acc_sh.at[pl.ds(base_r + t * CH, CH)])
            return carry
        lax.fori_loop(0, nfull, _zcopy, 0)
        pltpu.sync_copy(rin.at[0].at[pl.ds(0, rem)],
                        acc_sh.at[pl.ds(base_r + nfull * CH, rem)])

        @pl.when(s == NS - 1)
        def _ztail():
            pltpu.sync_copy(rin.at[0].at[pl.ds(0, N_ - NS * RPT)],
                            acc_sh.at[pl.ds(NS * RPT, N_ - NS * RPT)])

        pltpu.sync_copy(src_hbm.at[s], src_v)
        pltpu.sync_copy(dst_hbm.at[s], dst_v)
        pltpu.sync_copy(val_hbm.at[s], val_v)
        plsc.subcore_barrier()

        for b in range(NB):
            g_start(b, b)
        for b in range(NB):
            g_wait(b, b)
            mult(b, b)
            s_start(b, b)
            g_start(b, b + NB)

        def _round(g, carry):
            for b in range(NB):
                k = g * NB + b
                g_wait(b, k)
                s_wait(b, k - NB)
                mult(b, k)
                s_start(b, k)
                g_start(b, k + NB)
            return carry
        lax.fori_loop(1, NR - 1, _round, 0)

        for b in range(NB):
            k = (NR - 1) * NB + b
            g_wait(b, k)
            s_wait(b, k - NB)
            mult(b, k)
            s_start(b, k)
        for b in range(NB):
            s_wait(b, (NR - 1) * NB + b)
        plsc.subcore_barrier()

        pltpu.sync_copy(acc_sh.at[pl.ds(base_r, RPT)],
                        out_hbm.at[c, pl.ds(base_r, RPT)])

        @pl.when(s == NS - 1)
        def _otail():
            pltpu.sync_copy(acc_sh.at[pl.ds(NS * RPT, N_ - NS * RPT)],
                            out_hbm.at[c, pl.ds(NS * RPT, N_ - NS * RPT)])

    return prop(ego_cols, src3, dst3, val3)


def _update(p, ego_cols, acc, a_row, scale):
    blk = 1000

    def body(p_ref, e_ref, ac_ref, a_ref, eo_ref, ao_ref):
        a = a_ref[...]
        neigh = jnp.concatenate([p_ref[0], p_ref[1]], axis=1)
        ego = jnp.concatenate([e_ref[0], e_ref[1]], axis=1)
        new = jnp.maximum(a * neigh + (1.0 - a) * ego, 0.0)
        eo_ref[0] = new[:, :DH]
        eo_ref[1] = new[:, DH:]
        ao_ref[...] = (ac_ref[...] + new) * scale

    return pl.pallas_call(
        body,
        grid=(N_ // blk,),
        in_specs=[
            pl.BlockSpec((NC, blk, DH), lambda i: (0, i, 0)),
            pl.BlockSpec((NC, blk, DH), lambda i: (0, i, 0)),
            pl.BlockSpec((blk, D_), lambda i: (i, 0)),
            pl.BlockSpec((1, D_), lambda i: (0, 0)),
        ],
        out_specs=[
            pl.BlockSpec((NC, blk, DH), lambda i: (0, i, 0)),
            pl.BlockSpec((blk, D_), lambda i: (i, 0)),
        ],
        out_shape=[
            jax.ShapeDtypeStruct((NC, N_, DH), jnp.float32),
            jax.ShapeDtypeStruct((N_, D_), jnp.float32),
        ],
    )(p, ego_cols, acc, a_row)


def _gather7(allm, allm1, allm2, users, items_off, neg_off):
    RG = B_ // NW

    @functools.partial(
        pl.kernel,
        out_type=tuple(jax.ShapeDtypeStruct((B_, D_), jnp.float32)
                       for _ in range(7)),
        mesh=_sc_mesh(),
        scratch_types=[
            pltpu.VMEM((RG,), jnp.int32),
            pltpu.VMEM((RG,), jnp.int32),
            pltpu.VMEM((RG,), jnp.int32),
            pltpu.VMEM((RG, D_), jnp.float32),
        ],
    )
    def gk(t0, t1, t2, u_hbm, i_hbm, n_hbm,
           o_ue, o_ie, o_ine, o_u1, o_i1, o_u2, o_i2,
           iu, ii, inn, rows):
        c = lax.axis_index("c")
        s = lax.axis_index("s")
        b = (s * NC + c) * RG
        pltpu.sync_copy(u_hbm.at[pl.ds(b, RG)], iu)
        pltpu.sync_copy(i_hbm.at[pl.ds(b, RG)], ii)
        pltpu.sync_copy(n_hbm.at[pl.ds(b, RG)], inn)
        for tab, idx, out in ((t0, iu, o_ue), (t0, ii, o_ie), (t0, inn, o_ine),
                              (t1, iu, o_u1), (t1, ii, o_i1),
                              (t2, iu, o_u2), (t2, ii, o_i2)):
            pltpu.sync_copy(tab.at[idx], rows)
            pltpu.sync_copy(rows, out.at[pl.ds(b, RG)])

    return gk(allm, allm1, allm2, users, items_off, neg_off)


def _norm_rows(x):
    return x / (jnp.sqrt(jnp.sum(x * x, axis=1, keepdims=True)) + 1e-12)


def _sup(gu, gi, gn):
    def body(u_ref, i_ref, n_ref, o_ref):
        u = u_ref[...]
        o_ref[...] = jnp.sum(u * (i_ref[...] - n_ref[...]), axis=1,
                             keepdims=True)
    return pl.pallas_call(
        body,
        out_shape=jax.ShapeDtypeStruct((B_, 1), jnp.float32),
    )(gu, gi, gn)


def _ssl(g1, g2, tab):
    t_rows = tab.shape[0]

    def body(g1_ref, g2_ref, t_ref, o_ref):
        z1 = _norm_rows(g1_ref[...])
        z2 = _norm_rows(g2_ref[...])
        pos = jnp.sum(z1 * z2, axis=1, keepdims=True)
        tn = _norm_rows(t_ref[...])
        o_ref[...] = lax.dot_general(
            z1, tn, (((1,), (1,)), ((), ())),
            preferred_element_type=jnp.float32) - pos

    return pl.pallas_call(
        body,
        out_shape=jax.ShapeDtypeStruct((B_, t_rows), jnp.float32),
    )(g1, g2, tab)


def _nce(gu1, gu2, gi1, gi2):
    def one(a, b):
        z1 = _norm_rows(a)
        z2 = _norm_rows(b)
        neg = lax.dot_general(z1, z2, (((1,), (1,)), ((), ())),
                              preferred_element_type=jnp.float32) / TEMP
        pos = jnp.sum(z1 * z2, axis=1) / TEMP
        m = jnp.max(neg, axis=1)
        lse = jnp.log(jnp.sum(jnp.exp(neg - m[:, None]), axis=1)) + m
        return jnp.mean(lse - pos)

    def body(u1_ref, u2_ref, i1_ref, i2_ref, o_ref):
        o_ref[...] = jnp.reshape(
            one(u1_ref[...], u2_ref[...]) + one(i1_ref[...], i2_ref[...]),
            (1, 1))

    return pl.pallas_call(
        body,
        out_shape=jax.ShapeDtypeStruct((1, 1), jnp.float32),
    )(gu1, gu2, gi1, gi2)


def kernel(syn_embeddings, herb_embeddings, norm_adj_idx, norm_adj_val,
           sub1_idx, sub1_val, sub2_idx, sub2_val,
           users, items, neg_items, alpha):
    ego0 = jnp.concatenate([syn_embeddings, herb_embeddings], axis=0)
    ego0_cols = jnp.stack([ego0[:, :DH], ego0[:, DH:]], axis=0)
    a = jax.nn.sigmoid(alpha).astype(jnp.float32)
    a_row = jnp.broadcast_to(a, (1, D_))

    def run_graph(idx, val):
        src3 = idx[1].reshape(NS, NCH, CH)
        dst3 = idx[0].reshape(NS, NCH, CH)
        val3 = val.reshape(NS, NCH, CH)
        ego = ego0_cols
        acc = ego0
        for layer in range(3):
            p = _propagate(ego, src3, dst3, val3)
            scale = 0.25 if layer == 2 else 1.0
            ego, acc = _update(p, ego, acc, a_row, scale)
        return acc

    allm = run_graph(norm_adj_idx, norm_adj_val)
    allm1 = run_graph(sub1_idx, sub1_val)
    allm2 = run_graph(sub2_idx, sub2_val)

    items_off = items + SS_
    neg_off = neg_items + SS_
    g_ue, g_ie, g_ine, g_u1, g_i1, g_u2, g_i2 = _gather7(
        allm, allm1, allm2, users, items_off, neg_off)

    sup_logits = _sup(g_ue, g_ie, g_ine)[:, 0]
    ssl_logits_user = _ssl(g_u1, g_u2, allm2[:SS_])
    ssl_logits_item = _ssl(g_i1, g_i2, allm2[SS_:])
    infonce_loss = _nce(g_u1, g_u2, g_i1, g_i2)[0, 0]

    ue = allm[:SS_]
    ie = allm[SS_:]
    return (sup_logits, ssl_logits_user, ssl_logits_item, ue, ie,
            infonce_loss)
